# baseline (speedup 1.0000x reference)
import functools

import jax
import jax.numpy as jnp
from jax import lax
from jax.experimental import pallas as pl
from jax.experimental.pallas import tpu as pltpu
from jax.experimental.pallas import tpu_sc as plsc


def _consts_body(scal_ref, w1c_ref, w2_ref, w3_ref, wo_ref, b3c_ref, out_ref):
    w1c = w1c_ref[...]
    p = jnp.maximum(w1c, 0.0)
    n = jnp.minimum(w1c, 0.0)
    up = jnp.dot(w2_ref[...], p, preferred_element_type=jnp.float32)
    un = jnp.dot(w2_ref[...], n, preferred_element_type=jnp.float32)
    vp = jnp.dot(w3_ref[...], jnp.maximum(up, 0.0),
                 preferred_element_type=jnp.float32)
    vn = jnp.dot(w3_ref[...], jnp.minimum(un, 0.0),
                 preferred_element_type=jnp.float32)
    cp = jnp.dot(wo_ref[...], vp, preferred_element_type=jnp.float32)
    cn = jnp.dot(wo_ref[...], vn, preferred_element_type=jnp.float32)
    d0 = jnp.dot(wo_ref[...], b3c_ref[...],
                 preferred_element_type=jnp.float32) + scal_ref[4]
    def s(i):
        return jnp.full((1, 1), scal_ref[i], jnp.float32)
    out_ref[...] = jnp.concatenate(
        [cp, cn, d0, s(0), s(1), s(2), s(3), jnp.zeros((1, 9), jnp.float32)],
        axis=1)


def _tc_consts(scal, w1, w2, w3, wo, b3c):
    return pl.pallas_call(
        _consts_body,
        in_specs=[
            pl.BlockSpec(memory_space=pltpu.SMEM),
            pl.BlockSpec((1024, 1), lambda: (0, 0)),
            pl.BlockSpec((512, 1024), lambda: (0, 0)),
            pl.BlockSpec((256, 512), lambda: (0, 0)),
            pl.BlockSpec((1, 256), lambda: (0, 0)),
            pl.BlockSpec((256, 1), lambda: (0, 0)),
        ],
        out_specs=pl.BlockSpec((1, 16), lambda: (0, 0)),
        out_shape=jax.ShapeDtypeStruct((1, 16), jnp.float32),
    )(scal, w1, w2, w3, wo, b3c)


_L = 16
_RCHUNK = 8
_GCHUNK = 128


def _sc_all(idx, table_flat, consts):
    info = plsc.get_sparse_core_info()
    nc, ns = info.num_cores, info.num_subcores
    nw = nc * ns
    b = idx.shape[0]
    rpw = b // nw
    nchunk = rpw // _RCHUNK
    row_w = b * _RCHUNK
    mesh = plsc.VectorSubcoreMesh(core_axis_name="c", subcore_axis_name="s")

    @functools.partial(
        pl.kernel,
        mesh=mesh,
        out_type=jax.ShapeDtypeStruct((b * b,), jnp.float32),
        scratch_types=[
            pltpu.VMEM((_L,), jnp.float32),
            pltpu.VMEM((b,), jnp.int32),
            pltpu.VMEM((b,), jnp.float32),
            pltpu.VMEM((b,), jnp.float32),
            pltpu.VMEM((b,), jnp.float32),
            pltpu.VMEM((row_w,), jnp.float32),
            pltpu.VMEM((row_w,), jnp.float32),
            pltpu.SemaphoreType.DMA,
            pltpu.SemaphoreType.DMA,
        ],
    )
    def k(idx_hbm, table_hbm, c_hbm, out_hbm, cv_v, idx_v, e_v, sig_v,
          lin_v, buf0, buf1, gsem, wsem):
        wid = lax.axis_index("s") * nc + lax.axis_index("c")
        base = wid * rpw
        pltpu.sync_copy(c_hbm, cv_v)
        pltpu.sync_copy(idx_hbm, idx_v)
        gathers = [
            pltpu.make_async_copy(
                table_hbm.at[idx_v.at[pl.ds(g * _GCHUNK, _GCHUNK)]],
                e_v.at[pl.ds(g * _GCHUNK, _GCHUNK)], gsem)
            for g in range(b // _GCHUNK)
        ]
        for g in gathers:
            g.start()
        for g in gathers:
            g.wait()

        cv = cv_v[...]

        def splat(i):
            return cv.at[jnp.full((_L,), i, jnp.int32)].get(
                mode="promise_in_bounds")

        cp = splat(0)
        cn = splat(1)
        d0 = splat(2)
        w0s = splat(3)
        b0s = splat(4)
        wls = splat(5)
        bls = splat(6)

        def sig_chunk(kk, carry):
            for u in range(16):
                off = kk * 256 + u * _L
                ev = e_v[pl.ds(off, _L)]
                csel = jnp.where(ev >= 0.0, cp, cn)
                d = jnp.maximum(ev * csel + d0, 0.0)
                lg = d * wls + bls
                sig_v[pl.ds(off, _L)] = 1.0 / (1.0 + jnp.exp(-lg))
                lin_v[pl.ds(off, _L)] = ev * w0s + b0s
            return carry
        lax.fori_loop(0, b // 256, sig_chunk, 0)

        bufs = (buf0, buf1)

        def fill(buf, kk):
            for r in range(_RCHUNK):
                row = base + kk * _RCHUNK + r
                lvec = lin_v[pl.ds((row // _L) * _L, _L)]
                rv = lvec.at[jnp.full((_L,), row % _L, jnp.int32)].get(
                    mode="promise_in_bounds")

                def cols(c8, carry):
                    for u in range(32):
                        off = c8 * 512 + u * _L
                        buf[pl.ds(r * b + off, _L)] = (
                            sig_v[pl.ds(off, _L)] + rv)
                    return carry
                lax.fori_loop(0, b // 512, cols, 0)

        def wcopy(buf, kk):
            return pltpu.make_async_copy(
                buf, out_hbm.at[pl.ds((base + kk * _RCHUNK) * b, row_w)],
                wsem)

        def outer(k2, carry):
            for half in range(2):
                kk = k2 * 2 + half

                @pl.when(k2 > 0)
                def _drain():
                    wcopy(bufs[half], kk).wait()

                fill(bufs[half], kk)
                wcopy(bufs[half], kk).start()
            return carry
        lax.fori_loop(0, nchunk // 2, outer, 0)
        wcopy(buf0, nchunk - 2).wait()
        wcopy(buf1, nchunk - 1).wait()

    return k(idx, table_flat, consts)


def kernel(x, table, w0, b0, W1, b1, W2, b2, W3, b3, Wo, bo, Wl, bl):
    b = x.shape[0]
    idx = x.reshape(b).astype(jnp.int32)
    scal = jnp.stack(
        [w0[0, 0], b0[0], Wl[0, 0], bl[0], bo[0]]).astype(jnp.float32)
    consts = _tc_consts(scal, W1, W2, W3, Wo, b3.reshape(256, 1))
    out_flat = _sc_all(idx, table.reshape(-1).astype(jnp.float32),
                       consts.reshape(-1))
    return out_flat.reshape(b, b)


# speedup vs baseline: 1.0386x; 1.0386x over previous
import functools

import jax
import jax.numpy as jnp
from jax import lax
from jax.experimental import pallas as pl
from jax.experimental.pallas import tpu as pltpu
from jax.experimental.pallas import tpu_sc as plsc


def _consts_body(scal_ref, w1c_ref, w2_ref, w3_ref, wo_ref, b3c_ref, out_ref):
    w1c = w1c_ref[...]
    p = jnp.maximum(w1c, 0.0)
    n = jnp.minimum(w1c, 0.0)
    up = jnp.dot(w2_ref[...], p, preferred_element_type=jnp.float32)
    un = jnp.dot(w2_ref[...], n, preferred_element_type=jnp.float32)
    vp = jnp.dot(w3_ref[...], jnp.maximum(up, 0.0),
                 preferred_element_type=jnp.float32)
    vn = jnp.dot(w3_ref[...], jnp.minimum(un, 0.0),
                 preferred_element_type=jnp.float32)
    cp = jnp.dot(wo_ref[...], vp, preferred_element_type=jnp.float32)
    cn = jnp.dot(wo_ref[...], vn, preferred_element_type=jnp.float32)
    d0 = jnp.dot(wo_ref[...], b3c_ref[...],
                 preferred_element_type=jnp.float32) + scal_ref[4]
    def s(i):
        return jnp.full((1, 1), scal_ref[i], jnp.float32)
    out_ref[...] = jnp.concatenate(
        [cp, cn, d0, s(0), s(1), s(2), s(3), jnp.zeros((1, 9), jnp.float32)],
        axis=1)


def _tc_consts(scal, w1, w2, w3, wo, b3c):
    return pl.pallas_call(
        _consts_body,
        in_specs=[
            pl.BlockSpec(memory_space=pltpu.SMEM),
            pl.BlockSpec((1024, 1), lambda: (0, 0)),
            pl.BlockSpec((512, 1024), lambda: (0, 0)),
            pl.BlockSpec((256, 512), lambda: (0, 0)),
            pl.BlockSpec((1, 256), lambda: (0, 0)),
            pl.BlockSpec((256, 1), lambda: (0, 0)),
        ],
        out_specs=pl.BlockSpec((1, 16), lambda: (0, 0)),
        out_shape=jax.ShapeDtypeStruct((1, 16), jnp.float32),
    )(scal, w1, w2, w3, wo, b3c)


_L = 16
_RCHUNK = 8


def _sc_gather(idx, table_flat):
    info = plsc.get_sparse_core_info()
    nc, ns = info.num_cores, info.num_subcores
    nw = nc * ns
    b = idx.shape[0]
    bpw = b // nw
    mesh = plsc.VectorSubcoreMesh(core_axis_name="c", subcore_axis_name="s")

    @functools.partial(
        pl.kernel,
        mesh=mesh,
        out_type=jax.ShapeDtypeStruct((b,), jnp.float32),
        scratch_types=[
            pltpu.VMEM((bpw,), jnp.int32),
            pltpu.VMEM((bpw,), jnp.float32),
            pltpu.SemaphoreType.DMA,
        ],
    )
    def gather_kernel(idx_hbm, table_hbm, out_hbm, idx_v, rows_v, sem):
        wid = lax.axis_index("s") * nc + lax.axis_index("c")
        base = wid * bpw
        pltpu.sync_copy(idx_hbm.at[pl.ds(base, bpw)], idx_v)
        pltpu.async_copy(table_hbm.at[idx_v], rows_v, sem).wait()
        pltpu.sync_copy(rows_v, out_hbm.at[pl.ds(base, bpw)])

    return gather_kernel(idx, table_flat)


def _sc_broadcast(e, consts):
    info = plsc.get_sparse_core_info()
    nc, ns = info.num_cores, info.num_subcores
    nw = nc * ns
    b = e.shape[0]
    rpw = b // nw
    nchunk = rpw // _RCHUNK
    row_w = b * _RCHUNK
    mesh = plsc.VectorSubcoreMesh(core_axis_name="c", subcore_axis_name="s")

    @functools.partial(
        pl.kernel,
        mesh=mesh,
        out_type=jax.ShapeDtypeStruct((b * b,), jnp.float32),
        scratch_types=[
            pltpu.VMEM((_L,), jnp.float32),
            pltpu.VMEM((b,), jnp.float32),
            pltpu.VMEM((b,), jnp.float32),
            pltpu.VMEM((b,), jnp.float32),
            pltpu.VMEM((row_w,), jnp.float32),
            pltpu.VMEM((row_w,), jnp.float32),
            pltpu.SemaphoreType.DMA,
        ],
    )
    def k(e_hbm, c_hbm, out_hbm, cv_v, e_v, sig_v,
          lin_v, buf0, buf1, wsem):
        wid = lax.axis_index("s") * nc + lax.axis_index("c")
        base = wid * rpw
        pltpu.sync_copy(c_hbm, cv_v)
        pltpu.sync_copy(e_hbm, e_v)

        cv = cv_v[...]

        def splat(i):
            return cv.at[jnp.full((_L,), i, jnp.int32)].get(
                mode="promise_in_bounds")

        cp = splat(0)
        cn = splat(1)
        d0 = splat(2)
        w0s = splat(3)
        b0s = splat(4)
        wls = splat(5)
        bls = splat(6)

        def sig_chunk(kk, carry):
            for u in range(16):
                off = kk * 256 + u * _L
                ev = e_v[pl.ds(off, _L)]
                csel = jnp.where(ev >= 0.0, cp, cn)
                d = jnp.maximum(ev * csel + d0, 0.0)
                lg = d * wls + bls
                sig_v[pl.ds(off, _L)] = 1.0 / (1.0 + jnp.exp(-lg))
                lin_v[pl.ds(off, _L)] = ev * w0s + b0s
            return carry
        lax.fori_loop(0, b // 256, sig_chunk, 0)

        bufs = (buf0, buf1)

        def fill(buf, kk):
            for r in range(_RCHUNK):
                row = base + kk * _RCHUNK + r
                lvec = lin_v[pl.ds((row // _L) * _L, _L)]
                rv = lvec.at[jnp.full((_L,), row % _L, jnp.int32)].get(
                    mode="promise_in_bounds")

                def cols(c8, carry):
                    for u in range(32):
                        off = c8 * 512 + u * _L
                        buf[pl.ds(r * b + off, _L)] = (
                            sig_v[pl.ds(off, _L)] + rv)
                    return carry
                lax.fori_loop(0, b // 512, cols, 0)

        def wcopy(buf, kk):
            return pltpu.make_async_copy(
                buf, out_hbm.at[pl.ds((base + kk * _RCHUNK) * b, row_w)],
                wsem)

        def outer(k2, carry):
            for half in range(2):
                kk = k2 * 2 + half

                @pl.when(k2 > 0)
                def _drain():
                    wcopy(bufs[half], kk).wait()

                fill(bufs[half], kk)
                wcopy(bufs[half], kk).start()
            return carry
        lax.fori_loop(0, nchunk // 2, outer, 0)
        wcopy(buf0, nchunk - 2).wait()
        wcopy(buf1, nchunk - 1).wait()

    return k(e, consts)


def kernel(x, table, w0, b0, W1, b1, W2, b2, W3, b3, Wo, bo, Wl, bl):
    b = x.shape[0]
    idx = x.reshape(b).astype(jnp.int32)
    scal = jnp.stack(
        [w0[0, 0], b0[0], Wl[0, 0], bl[0], bo[0]]).astype(jnp.float32)
    consts = _tc_consts(scal, W1, W2, W3, Wo, b3.reshape(256, 1))
    e = _sc_gather(idx, table.reshape(-1).astype(jnp.float32))
    out_flat = _sc_broadcast(e, consts.reshape(-1))
    return out_flat.reshape(b, b)


# speedup vs baseline: 1.0689x; 1.0292x over previous
import functools

import jax
import jax.numpy as jnp
from jax import lax
from jax.experimental import pallas as pl
from jax.experimental.pallas import tpu as pltpu
from jax.experimental.pallas import tpu_sc as plsc


def _consts_body(scal_ref, w1c_ref, w2_ref, w3_ref, wo_ref, b3c_ref, out_ref):
    w1c = w1c_ref[...]
    p = jnp.maximum(w1c, 0.0)
    n = jnp.minimum(w1c, 0.0)
    up = jnp.dot(w2_ref[...], p, preferred_element_type=jnp.float32)
    un = jnp.dot(w2_ref[...], n, preferred_element_type=jnp.float32)
    vp = jnp.dot(w3_ref[...], jnp.maximum(up, 0.0),
                 preferred_element_type=jnp.float32)
    vn = jnp.dot(w3_ref[...], jnp.minimum(un, 0.0),
                 preferred_element_type=jnp.float32)
    cp = jnp.dot(wo_ref[...], vp, preferred_element_type=jnp.float32)
    cn = jnp.dot(wo_ref[...], vn, preferred_element_type=jnp.float32)
    d0 = jnp.dot(wo_ref[...], b3c_ref[...],
                 preferred_element_type=jnp.float32) + scal_ref[4]
    def s(i):
        return jnp.full((1, 1), scal_ref[i], jnp.float32)
    out_ref[...] = jnp.concatenate(
        [cp, cn, d0, s(0), s(1), s(2), s(3), jnp.zeros((1, 9), jnp.float32)],
        axis=1)


def _tc_consts(scal, w1, w2, w3, wo, b3c):
    return pl.pallas_call(
        _consts_body,
        in_specs=[
            pl.BlockSpec(memory_space=pltpu.SMEM),
            pl.BlockSpec((1024, 1), lambda: (0, 0)),
            pl.BlockSpec((512, 1024), lambda: (0, 0)),
            pl.BlockSpec((256, 512), lambda: (0, 0)),
            pl.BlockSpec((1, 256), lambda: (0, 0)),
            pl.BlockSpec((256, 1), lambda: (0, 0)),
        ],
        out_specs=pl.BlockSpec((1, 16), lambda: (0, 0)),
        out_shape=jax.ShapeDtypeStruct((1, 16), jnp.float32),
    )(scal, w1, w2, w3, wo, b3c)


_L = 16
_RCHUNK = 8


def _sc_gather(idx, table_flat):
    info = plsc.get_sparse_core_info()
    nc, ns = info.num_cores, info.num_subcores
    nw = nc * ns
    b = idx.shape[0]
    bpw = b // nw
    mesh = plsc.VectorSubcoreMesh(core_axis_name="c", subcore_axis_name="s")

    @functools.partial(
        pl.kernel,
        mesh=mesh,
        out_type=jax.ShapeDtypeStruct((b,), jnp.float32),
        scratch_types=[
            pltpu.VMEM((bpw,), jnp.int32),
            pltpu.VMEM((bpw,), jnp.float32),
            pltpu.SemaphoreType.DMA,
        ],
    )
    def gather_kernel(idx_hbm, table_hbm, out_hbm, idx_v, rows_v, sem):
        wid = lax.axis_index("s") * nc + lax.axis_index("c")
        base = wid * bpw
        pltpu.sync_copy(idx_hbm.at[pl.ds(base, bpw)], idx_v)
        pltpu.async_copy(table_hbm.at[idx_v], rows_v, sem).wait()
        pltpu.sync_copy(rows_v, out_hbm.at[pl.ds(base, bpw)])

    return gather_kernel(idx, table_flat)


def _sc_broadcast(e, consts):
    info = plsc.get_sparse_core_info()
    nc, ns = info.num_cores, info.num_subcores
    nw = nc * ns
    b = e.shape[0]
    rpw = b // nw
    nchunk = rpw // _RCHUNK
    row_w = b * _RCHUNK
    mesh = plsc.VectorSubcoreMesh(core_axis_name="c", subcore_axis_name="s")

    @functools.partial(
        pl.kernel,
        mesh=mesh,
        out_type=jax.ShapeDtypeStruct((b * b,), jnp.float32),
        scratch_types=[
            pltpu.VMEM((_L,), jnp.float32),
            pltpu.VMEM((b,), jnp.float32),
            pltpu.VMEM((b,), jnp.float32),
            pltpu.VMEM((b,), jnp.float32),
            pltpu.VMEM((b,), jnp.float32),
            pltpu.VMEM((b,), jnp.float32),
            pltpu.VMEM((b,), jnp.float32),
            pltpu.VMEM((b,), jnp.float32),
            pltpu.SemaphoreType.DMA,
        ],
    )
    def k(e_hbm, c_hbm, out_hbm, cv_v, e_v, sig_v,
          lin_v, buf0, buf1, buf2, buf3, wsem):
        wid = lax.axis_index("s") * nc + lax.axis_index("c")
        base = wid * rpw
        pltpu.sync_copy(c_hbm, cv_v)
        pltpu.sync_copy(e_hbm, e_v)

        cv = cv_v[...]

        def splat(i):
            return cv.at[jnp.full((_L,), i, jnp.int32)].get(
                mode="promise_in_bounds")

        cp = splat(0)
        cn = splat(1)
        d0 = splat(2)
        w0s = splat(3)
        b0s = splat(4)
        wls = splat(5)
        bls = splat(6)

        def sig_chunk(kk, carry):
            for u in range(16):
                off = kk * 256 + u * _L
                ev = e_v[pl.ds(off, _L)]
                csel = jnp.where(ev >= 0.0, cp, cn)
                d = jnp.maximum(ev * csel + d0, 0.0)
                lg = d * wls + bls
                sig_v[pl.ds(off, _L)] = 1.0 / (1.0 + jnp.exp(-lg))
                lin_v[pl.ds(off, _L)] = ev * w0s + b0s
            return carry
        lax.fori_loop(0, b // 256, sig_chunk, 0)

        bufs = (buf0, buf1, buf2, buf3)
        nring = len(bufs)

        def wcopy(buf, row_local):
            return pltpu.make_async_copy(
                buf, out_hbm.at[pl.ds((base + row_local) * b, b)], wsem)

        def outer(g, carry):
            for nb in range(nring):
                row_local = g * nring + nb
                row = base + row_local

                @pl.when(g > 0)
                def _drain():
                    wcopy(bufs[nb], row_local).wait()

                lvec = lin_v[pl.ds((row // _L) * _L, _L)]
                rv = lvec.at[jnp.full((_L,), row % _L, jnp.int32)].get(
                    mode="promise_in_bounds")
                for u in range(b // _L):
                    bufs[nb][pl.ds(u * _L, _L)] = sig_v[pl.ds(u * _L, _L)] + rv
                wcopy(bufs[nb], row_local).start()
            return carry
        lax.fori_loop(0, rpw // nring, outer, 0)
        for nb in range(nring):
            wcopy(bufs[nb], 0).wait()

    return k(e, consts)


def kernel(x, table, w0, b0, W1, b1, W2, b2, W3, b3, Wo, bo, Wl, bl):
    b = x.shape[0]
    idx = x.reshape(b).astype(jnp.int32)
    scal = jnp.stack(
        [w0[0, 0], b0[0], Wl[0, 0], bl[0], bo[0]]).astype(jnp.float32)
    consts = _tc_consts(scal, W1, W2, W3, Wo, b3.reshape(256, 1))
    e = _sc_gather(idx, table.reshape(-1).astype(jnp.float32))
    out_flat = _sc_broadcast(e, consts.reshape(-1))
    return out_flat.reshape(b, b)


# speedup vs baseline: 1.8871x; 1.7654x over previous
import functools

import jax
import jax.numpy as jnp
from jax import lax
from jax.experimental import pallas as pl
from jax.experimental.pallas import tpu as pltpu
from jax.experimental.pallas import tpu_sc as plsc


def _consts_body(scal_ref, w1c_ref, w2_ref, w3_ref, wo_ref, b3c_ref, out_ref):
    w1c = w1c_ref[...]
    p = jnp.maximum(w1c, 0.0)
    n = jnp.minimum(w1c, 0.0)
    up = jnp.dot(w2_ref[...], p, preferred_element_type=jnp.float32)
    un = jnp.dot(w2_ref[...], n, preferred_element_type=jnp.float32)
    vp = jnp.dot(w3_ref[...], jnp.maximum(up, 0.0),
                 preferred_element_type=jnp.float32)
    vn = jnp.dot(w3_ref[...], jnp.minimum(un, 0.0),
                 preferred_element_type=jnp.float32)
    cp = jnp.dot(wo_ref[...], vp, preferred_element_type=jnp.float32)
    cn = jnp.dot(wo_ref[...], vn, preferred_element_type=jnp.float32)
    d0 = jnp.dot(wo_ref[...], b3c_ref[...],
                 preferred_element_type=jnp.float32) + scal_ref[4]
    def s(i):
        return jnp.full((1, 1), scal_ref[i], jnp.float32)
    out_ref[...] = jnp.concatenate(
        [cp, cn, d0, s(0), s(1), s(2), s(3), jnp.zeros((1, 9), jnp.float32)],
        axis=1)


def _tc_consts(scal, w1, w2, w3, wo, b3c):
    return pl.pallas_call(
        _consts_body,
        in_specs=[
            pl.BlockSpec(memory_space=pltpu.SMEM),
            pl.BlockSpec((1024, 1), lambda: (0, 0)),
            pl.BlockSpec((512, 1024), lambda: (0, 0)),
            pl.BlockSpec((256, 512), lambda: (0, 0)),
            pl.BlockSpec((1, 256), lambda: (0, 0)),
            pl.BlockSpec((256, 1), lambda: (0, 0)),
        ],
        out_specs=pl.BlockSpec((1, 16), lambda: (0, 0)),
        out_shape=jax.ShapeDtypeStruct((1, 16), jnp.float32),
    )(scal, w1, w2, w3, wo, b3c)


_L = 16
_RCHUNK = 8


def _sc_gather(idx, table_flat):
    info = plsc.get_sparse_core_info()
    nc, ns = info.num_cores, info.num_subcores
    nw = nc * ns
    b = idx.shape[0]
    bpw = b // nw
    mesh = plsc.VectorSubcoreMesh(core_axis_name="c", subcore_axis_name="s")

    @functools.partial(
        pl.kernel,
        mesh=mesh,
        out_type=jax.ShapeDtypeStruct((b,), jnp.float32),
        scratch_types=[
            pltpu.VMEM((bpw,), jnp.int32),
            pltpu.VMEM((bpw,), jnp.float32),
            pltpu.SemaphoreType.DMA,
        ],
    )
    def gather_kernel(idx_hbm, table_hbm, out_hbm, idx_v, rows_v, sem):
        wid = lax.axis_index("s") * nc + lax.axis_index("c")
        base = wid * bpw
        pltpu.sync_copy(idx_hbm.at[pl.ds(base, bpw)], idx_v)
        pltpu.async_copy(table_hbm.at[idx_v], rows_v, sem).wait()
        pltpu.sync_copy(rows_v, out_hbm.at[pl.ds(base, bpw)])

    return gather_kernel(idx, table_flat)


def _bcast_body(cs_ref, e_row_ref, e_col_ref, out_ref, sig_ref):
    j = pl.program_id(0)

    @pl.when(j == 0)
    def _sig():
        ev = e_row_ref[...]
        csel = jnp.where(ev >= 0.0, cs_ref[0], cs_ref[1])
        d = jnp.maximum(ev * csel + cs_ref[2], 0.0)
        lg = d * cs_ref[5] + cs_ref[6]
        sig_ref[...] = 1.0 / (1.0 + jnp.exp(-lg))

    lin = e_col_ref[...] * cs_ref[3] + cs_ref[4]
    out_ref[...] = lin + sig_ref[...]


def _tc_broadcast(e, consts):
    b = e.shape[0]
    rt = 512
    nrt = b // rt
    return pl.pallas_call(
        _bcast_body,
        grid=(nrt,),
        in_specs=[
            pl.BlockSpec(memory_space=pltpu.SMEM),
            pl.BlockSpec((1, b), lambda j: (0, 0)),
            pl.BlockSpec((rt, 1), lambda j: (j, 0)),
        ],
        out_specs=pl.BlockSpec((rt, b), lambda j: (j, 0)),
        out_shape=jax.ShapeDtypeStruct((b, b), jnp.float32),
        scratch_shapes=[pltpu.VMEM((1, b), jnp.float32)],
        compiler_params=pltpu.CompilerParams(
            dimension_semantics=("arbitrary",),
        ),
    )(consts, e.reshape(1, b), e.reshape(b, 1))


def kernel(x, table, w0, b0, W1, b1, W2, b2, W3, b3, Wo, bo, Wl, bl):
    b = x.shape[0]
    idx = x.reshape(b).astype(jnp.int32)
    scal = jnp.stack(
        [w0[0, 0], b0[0], Wl[0, 0], bl[0], bo[0]]).astype(jnp.float32)
    consts = _tc_consts(scal, W1, W2, W3, Wo, b3.reshape(256, 1))
    e = _sc_gather(idx, table.reshape(-1).astype(jnp.float32))
    return _tc_broadcast(e, consts.reshape(-1))


# speedup vs baseline: 1.9133x; 1.0139x over previous
import functools

import jax
import jax.numpy as jnp
from jax import lax
from jax.experimental import pallas as pl
from jax.experimental.pallas import tpu as pltpu
from jax.experimental.pallas import tpu_sc as plsc


def _consts_body(scal_ref, w1c_ref, w2_ref, w3_ref, wo_ref, b3c_ref, out_ref):
    w1c = w1c_ref[...]
    p = jnp.maximum(w1c, 0.0)
    n = jnp.minimum(w1c, 0.0)
    up = jnp.dot(w2_ref[...], p, preferred_element_type=jnp.float32)
    un = jnp.dot(w2_ref[...], n, preferred_element_type=jnp.float32)
    vp = jnp.dot(w3_ref[...], jnp.maximum(up, 0.0),
                 preferred_element_type=jnp.float32)
    vn = jnp.dot(w3_ref[...], jnp.minimum(un, 0.0),
                 preferred_element_type=jnp.float32)
    cp = jnp.dot(wo_ref[...], vp, preferred_element_type=jnp.float32)
    cn = jnp.dot(wo_ref[...], vn, preferred_element_type=jnp.float32)
    d0 = jnp.dot(wo_ref[...], b3c_ref[...],
                 preferred_element_type=jnp.float32) + scal_ref[4]
    def s(i):
        return jnp.full((1, 1), scal_ref[i], jnp.float32)
    out_ref[...] = jnp.concatenate(
        [cp, cn, d0, s(0), s(1), s(2), s(3), jnp.zeros((1, 9), jnp.float32)],
        axis=1)


def _tc_consts(scal, w1, w2, w3, wo, b3c):
    return pl.pallas_call(
        _consts_body,
        in_specs=[
            pl.BlockSpec(memory_space=pltpu.SMEM),
            pl.BlockSpec((1024, 1), lambda: (0, 0)),
            pl.BlockSpec((512, 1024), lambda: (0, 0)),
            pl.BlockSpec((256, 512), lambda: (0, 0)),
            pl.BlockSpec((1, 256), lambda: (0, 0)),
            pl.BlockSpec((256, 1), lambda: (0, 0)),
        ],
        out_specs=pl.BlockSpec((1, 16), lambda: (0, 0)),
        out_shape=jax.ShapeDtypeStruct((1, 16), jnp.float32),
    )(scal, w1, w2, w3, wo, b3c)


_L = 16
_RCHUNK = 8


def _sc_gather(idx, table_flat):
    info = plsc.get_sparse_core_info()
    nc, ns = info.num_cores, info.num_subcores
    nw = nc * ns
    b = idx.shape[0]
    bpw = b // nw
    mesh = plsc.VectorSubcoreMesh(core_axis_name="c", subcore_axis_name="s")

    @functools.partial(
        pl.kernel,
        mesh=mesh,
        out_type=jax.ShapeDtypeStruct((b,), jnp.float32),
        scratch_types=[
            pltpu.VMEM((bpw,), jnp.int32),
            pltpu.VMEM((bpw,), jnp.float32),
            pltpu.SemaphoreType.DMA,
        ],
    )
    def gather_kernel(idx_hbm, table_hbm, out_hbm, idx_v, rows_v, sem):
        wid = lax.axis_index("s") * nc + lax.axis_index("c")
        base = wid * bpw
        pltpu.sync_copy(idx_hbm.at[pl.ds(base, bpw)], idx_v)
        pltpu.async_copy(table_hbm.at[idx_v], rows_v, sem).wait()
        pltpu.sync_copy(rows_v, out_hbm.at[pl.ds(base, bpw)])

    return gather_kernel(idx, table_flat)


def _bcast_body(scal_ref, e_row_ref, e_col_ref, w1c_ref, w2_ref, w3_ref,
                wo_ref, b3c_ref, out_ref, sig_ref):
    j = pl.program_id(0)

    @pl.when(j == 0)
    def _sig():
        w1c = w1c_ref[...]
        p = jnp.maximum(w1c, 0.0)
        n = jnp.minimum(w1c, 0.0)
        up = jnp.dot(w2_ref[...], p, preferred_element_type=jnp.float32)
        un = jnp.dot(w2_ref[...], n, preferred_element_type=jnp.float32)
        vp = jnp.dot(w3_ref[...], jnp.maximum(up, 0.0),
                     preferred_element_type=jnp.float32)
        vn = jnp.dot(w3_ref[...], jnp.minimum(un, 0.0),
                     preferred_element_type=jnp.float32)
        cp = jnp.dot(wo_ref[...], vp, preferred_element_type=jnp.float32)
        cn = jnp.dot(wo_ref[...], vn, preferred_element_type=jnp.float32)
        d0 = jnp.dot(wo_ref[...], b3c_ref[...],
                     preferred_element_type=jnp.float32) + scal_ref[4]
        ev = e_row_ref[...]
        csel = jnp.where(ev >= 0.0, cp, cn)
        d = jnp.maximum(ev * csel + d0, 0.0)
        lg = d * scal_ref[2] + scal_ref[3]
        sig_ref[...] = 1.0 / (1.0 + jnp.exp(-lg))

    lin = e_col_ref[...] * scal_ref[0] + scal_ref[1]
    out_ref[...] = lin + sig_ref[...]


def _tc_broadcast(e, scal, w1, w2, w3, wo, b3c):
    b = e.shape[0]
    rt = 512
    nrt = b // rt
    return pl.pallas_call(
        _bcast_body,
        grid=(nrt,),
        in_specs=[
            pl.BlockSpec(memory_space=pltpu.SMEM),
            pl.BlockSpec((1, b), lambda j: (0, 0)),
            pl.BlockSpec((rt, 1), lambda j: (j, 0)),
            pl.BlockSpec((1024, 1), lambda j: (0, 0)),
            pl.BlockSpec((512, 1024), lambda j: (0, 0)),
            pl.BlockSpec((256, 512), lambda j: (0, 0)),
            pl.BlockSpec((1, 256), lambda j: (0, 0)),
            pl.BlockSpec((256, 1), lambda j: (0, 0)),
        ],
        out_specs=pl.BlockSpec((rt, b), lambda j: (j, 0)),
        out_shape=jax.ShapeDtypeStruct((b, b), jnp.float32),
        scratch_shapes=[pltpu.VMEM((1, b), jnp.float32)],
        compiler_params=pltpu.CompilerParams(
            dimension_semantics=("arbitrary",),
        ),
    )(scal, e.reshape(1, b), e.reshape(b, 1), w1, w2, w3, wo, b3c)


def kernel(x, table, w0, b0, W1, b1, W2, b2, W3, b3, Wo, bo, Wl, bl):
    b = x.shape[0]
    idx = x.reshape(b).astype(jnp.int32)
    scal = jnp.stack(
        [w0[0, 0], b0[0], Wl[0, 0], bl[0], bo[0]]).astype(jnp.float32)
    e = _sc_gather(idx, table.reshape(-1).astype(jnp.float32))
    return _tc_broadcast(e, scal, W1, W2, W3, Wo, b3.reshape(256, 1))
